# baseline (speedup 1.0000x reference)
import jax
import jax.numpy as jnp
from jax import lax
from jax.experimental import pallas as pl
from jax.experimental.pallas import tpu as pltpu
from jax.experimental.pallas import tpu_sc as plsc

D = 256
BE = 128
CH = 512
LANES = 16
SENT = 1 << 28
N_WORD, N_TOPIC, N_DOC = 50000, 10000, 10000


def _round_up(x, m):
    return (x + m - 1) // m * m


def _bi32(s):
    return jnp.broadcast_to(s.astype(jnp.int32), (LANES,))


def _build_bin_kernel(e_pad, shift, n_passes, ncls, n_eff=0):
    ept = e_pad // 32
    nbins = n_passes * ncls
    cap = ept + CH + 16

    mesh = plsc.VectorSubcoreMesh(core_axis_name="c", subcore_axis_name="s")

    scratch = [
        pltpu.VMEM((ept,), jnp.int32),
        pltpu.VMEM((ept,), jnp.int32),
        pltpu.VMEM((ept,), jnp.float32),
        pltpu.VMEM((16,), jnp.int32),
    ]
    for _ in range(nbins):
        scratch += [
            pltpu.VMEM((cap,), jnp.int32),
            pltpu.VMEM((cap,), jnp.int32),
            pltpu.VMEM((cap,), jnp.float32),
        ]
    if ncls == 3:
        scratch.append(pltpu.VMEM((n_eff,), jnp.float32))
    scratch.append(pltpu.SemaphoreType.DMA)

    def body(*refs):
        nin = 3 + (1 if ncls == 3 else 0)
        src_hbm, dst_hbm, ew_hbm = refs[0:3]
        eff_hbm = refs[3] if ncls == 3 else None
        seg_src, seg_dst, seg_ew, counts_hbm = refs[nin:nin + 4]
        rest = refs[nin + 4:]
        src_v, dst_v, ew_v, counts_buf = rest[0:4]
        cmp = [rest[4 + 3 * b:7 + 3 * b] for b in range(nbins)]
        if ncls == 3:
            eff_v = rest[4 + 3 * nbins]
        sem = rest[-1]

        core = lax.axis_index("c")
        tile = lax.axis_index("s")
        wid = core * 16 + tile

        pltpu.sync_copy(src_hbm.at[pl.ds(wid * ept, ept)], src_v)
        pltpu.sync_copy(dst_hbm.at[pl.ds(wid * ept, ept)], dst_v)
        pltpu.sync_copy(ew_hbm.at[pl.ds(wid * ept, ept)], ew_v)
        if ncls == 3:
            pltpu.sync_copy(eff_hbm, eff_v)

        lane = lax.broadcasted_iota(jnp.int32, (LANES,), 0)
        fzero_v = jnp.full((LANES,), 0.0, jnp.float32)
        trash_v = jnp.full((LANES,), cap - 1, jnp.int32)
        sent_v = jnp.full((LANES,), SENT, jnp.int32)

        def scan_body(i, cnts):
            b = i * LANES
            sv = src_v[pl.ds(b, LANES)]
            dv = dst_v[pl.ds(b, LANES)]
            wv = ew_v[pl.ds(b, LANES)]
            pv = lax.shift_right_logical(dv, jnp.full((LANES,), shift,
                                                      jnp.int32))
            if ncls == 3:
                ev = plsc.load_gather(eff_v, [sv])
                clsv = jnp.where(ev > fzero_v, _bi32(jnp.int32(0)),
                                 jnp.where(ev < fzero_v,
                                           _bi32(jnp.int32(1)),
                                           _bi32(jnp.int32(2))))
                binv = pv * _bi32(jnp.int32(3)) + clsv
            else:
                binv = pv
            out = []
            for b_i in range(nbins):
                m = binv == _bi32(jnp.int32(b_i))
                mi = m.astype(jnp.int32)
                rank = plsc.cumsum(mi) - mi
                pos = jnp.where(m, _bi32(cnts[b_i]) + rank, trash_v)
                plsc.store_scatter(cmp[b_i][0], [pos], sv)
                plsc.store_scatter(cmp[b_i][1], [pos], dv)
                plsc.store_scatter(cmp[b_i][2], [pos], wv)
                out.append(cnts[b_i] + jnp.sum(mi))
            return tuple(out)

        cnts = lax.fori_loop(0, ept // LANES, scan_body,
                             tuple(jnp.int32(0) for _ in range(nbins)))

        for b_i in range(nbins):
            for k in range(CH // LANES):
                pos = _bi32(cnts[b_i]) + lane + k * LANES
                plsc.store_scatter(cmp[b_i][1], [pos], sent_v)
            plsc.store_scatter(counts_buf, [_bi32(jnp.int32(b_i))],
                               _bi32(cnts[b_i]))
            seg_off = (b_i * 32 + wid) * cap
            pltpu.sync_copy(cmp[b_i][0], seg_src.at[pl.ds(seg_off, cap)])
            pltpu.sync_copy(cmp[b_i][1], seg_dst.at[pl.ds(seg_off, cap)])
            pltpu.sync_copy(cmp[b_i][2], seg_ew.at[pl.ds(seg_off, cap)])
        pltpu.sync_copy(counts_buf, counts_hbm.at[pl.ds(wid * 16, 16)])

    out_type = [
        jax.ShapeDtypeStruct((nbins * 32 * cap,), jnp.int32),
        jax.ShapeDtypeStruct((nbins * 32 * cap,), jnp.int32),
        jax.ShapeDtypeStruct((nbins * 32 * cap,), jnp.float32),
        jax.ShapeDtypeStruct((512,), jnp.int32),
    ]
    return pl.kernel(body, out_type=out_type, mesh=mesh,
                     scratch_types=scratch,
                     compiler_params=pltpu.CompilerParams(
                         needs_layout_passes=False)), cap


def _build_acc_kernel(seg_cap, n_passes, ncls, R):
    n_dst_pad = n_passes * 32 * R
    mesh = plsc.VectorSubcoreMesh(core_axis_name="c", subcore_axis_name="s")

    ccap = CH + BE + 16
    scratch = [
        pltpu.VMEM((CH,), jnp.int32),
        pltpu.VMEM((CH,), jnp.int32),
        pltpu.VMEM((CH,), jnp.float32),
        pltpu.VMEM((ccap,), jnp.int32),
        pltpu.VMEM((ccap,), jnp.int32),
        pltpu.VMEM((ccap,), jnp.float32),
        pltpu.VMEM((BE,), jnp.int32),
        pltpu.VMEM((BE,), jnp.int32),
        pltpu.VMEM((BE, D), jnp.float32),
        pltpu.VMEM((512,), jnp.int32),
    ]
    for _ in range(ncls):
        scratch.append(pltpu.VMEM((R + 8, D), jnp.float32))
    for _ in range(ncls):
        scratch.append(pltpu.VMEM(((R + 8) * 16,), jnp.float32))
    scratch.append(pltpu.SemaphoreType.DMA)

    def body(*refs):
        x_hbm, seg_src, seg_dst, seg_ew, counts_hbm = refs[0:5]
        outs = refs[5:5 + 2 * ncls]
        rest = refs[5 + 2 * ncls:]
        (sbuf_src, sbuf_dst, sbuf_ew, cmp_src, cmp_dl, cmp_ew,
         src_blk, dl_blk, stage, counts_v) = rest[0:10]
        accs = rest[10:10 + ncls]
        auxs = rest[10 + ncls:10 + 2 * ncls]
        sem = rest[-1]

        core = lax.axis_index("c")
        tile = lax.axis_index("s")
        wid = core * 16 + tile

        pltpu.sync_copy(counts_hbm, counts_v)

        lane = lax.broadcasted_iota(jnp.int32, (LANES,), 0)
        blend0 = lane == 0
        blend1 = lane == 1
        ones_v = jnp.full((LANES,), 1.0, jnp.float32)
        fzero_v = jnp.full((LANES,), 0.0, jnp.float32)
        zero_iv = jnp.full((LANES,), 0, jnp.int32)
        trash_v = jnp.full((LANES,), ccap - 1, jnp.int32)
        dummy_v = jnp.full((LANES,), R, jnp.int32)
        cols = [lane + g * LANES for g in range(D // LANES)]

        def _scal(ref, idx):
            g = plsc.load_gather(ref, [_bi32(idx)])
            return jnp.sum(jnp.where(blend0, g, zero_iv))

        for p in range(n_passes):
            my_lo = (p * 32 + wid) * R
            lo_v = _bi32(my_lo)
            hi_v = _bi32(my_lo + R)

            def zero_body(r, _):
                for g in range(D // LANES):
                    accs_r = r
                    for c in range(ncls):
                        accs[c][accs_r, pl.ds(g * LANES, LANES)] = fzero_v
                for c in range(ncls):
                    auxs[c][pl.ds(r * LANES, LANES)] = fzero_v
                return 0

            lax.fori_loop(0, R, zero_body, 0)

            for cls in range(ncls):
                b_i = p * ncls + cls
                acc = accs[cls]
                aux = auxs[cls]

                def w_body(w, _, b_i=b_i, acc=acc, aux=aux,
                           lo_v=lo_v, hi_v=hi_v):
                    cnt = _scal(counts_v, w * 16 + jnp.int32(b_i))
                    nch = (cnt + CH) // CH

                    def ch_body(ch, _, w=w, b_i=b_i, acc=acc, aux=aux,
                                lo_v=lo_v, hi_v=hi_v):
                        off = (jnp.int32(b_i) * 32 + w) * seg_cap + ch * CH
                        pltpu.sync_copy(
                            seg_src.at[pl.ds(off, CH)], sbuf_src)
                        pltpu.sync_copy(
                            seg_dst.at[pl.ds(off, CH)], sbuf_dst)
                        pltpu.sync_copy(
                            seg_ew.at[pl.ds(off, CH)], sbuf_ew)

                        def scan_body(i, cnt2):
                            b = i * LANES
                            sv = sbuf_src[pl.ds(b, LANES)]
                            dv = sbuf_dst[pl.ds(b, LANES)]
                            wv = sbuf_ew[pl.ds(b, LANES)]
                            m = (dv >= lo_v) & (dv < hi_v)
                            mi = m.astype(jnp.int32)
                            rank = plsc.cumsum(mi) - mi
                            pos = jnp.where(m, _bi32(cnt2) + rank, trash_v)
                            plsc.store_scatter(cmp_src, [pos], sv)
                            plsc.store_scatter(cmp_dl, [pos], dv - lo_v)
                            plsc.store_scatter(cmp_ew, [pos], wv)
                            return cnt2 + jnp.sum(mi)

                        nm = lax.fori_loop(0, CH // LANES, scan_body,
                                           jnp.int32(0))
                        nblk = (nm + (BE - 1)) // BE

                        def blk_body(bi2, _, nm=nm, acc=acc, aux=aux):
                            base = bi2 * BE
                            for g in range(BE // LANES):
                                o2 = base + g * LANES
                                sv = cmp_src[pl.ds(o2, LANES)]
                                dv = cmp_dl[pl.ds(o2, LANES)]
                                valid = (lane + _bi32(o2)) < _bi32(nm)
                                src_blk[pl.ds(g * LANES, LANES)] = jnp.where(
                                    valid, sv, zero_iv)
                                dl_blk[pl.ds(g * LANES, LANES)] = jnp.where(
                                    valid, dv, dummy_v)
                            pltpu.async_copy(x_hbm.at[src_blk], stage,
                                             sem).wait()

                            def row_body(j, _):
                                rowv = plsc.load_gather(dl_blk, [_bi32(j)])
                                ewv = plsc.load_gather(
                                    cmp_ew, [_bi32(base + j)])
                                for g in range(D // LANES):
                                    v = stage[j, pl.ds(g * LANES, LANES)]
                                    plsc.addupdate_scatter(
                                        acc, [rowv, cols[g]], v * ewv)
                                av = jnp.where(
                                    blend0, ewv,
                                    jnp.where(blend1, ones_v, fzero_v))
                                plsc.addupdate_scatter(
                                    aux, [rowv * LANES + lane], av)
                                return 0

                            lax.fori_loop(0, BE, row_body, 0)
                            return 0

                        lax.fori_loop(0, nblk, blk_body, 0)
                        return 0

                    lax.fori_loop(0, nch, ch_body, 0)
                    return 0

                lax.fori_loop(0, 32, w_body, 0)

            for c in range(ncls):
                pltpu.sync_copy(accs[c].at[pl.ds(0, R)],
                                outs[2 * c].at[pl.ds(my_lo, R)])
                pltpu.sync_copy(auxs[c].at[pl.ds(0, R * 16)],
                                outs[2 * c + 1].at[pl.ds(my_lo * 16, R * 16)])

    out_type = []
    for _ in range(ncls):
        out_type += [jax.ShapeDtypeStruct((n_dst_pad, D), jnp.float32),
                     jax.ShapeDtypeStruct((n_dst_pad * 16,), jnp.float32)]
    return pl.kernel(body, out_type=out_type, mesh=mesh,
                     scratch_types=scratch,
                     compiler_params=pltpu.CompilerParams(
                         needs_layout_passes=False))


def _seg_reduce(x, src, dst, ew, E, shift, n_passes, ncls, R, eff=None):
    e_pad = _round_up(E, 512)
    p = e_pad - E
    src = jnp.pad(src, (0, p))
    dst = jnp.pad(dst, (0, p), constant_values=SENT)
    ew = jnp.pad(ew, (0, p))
    n_eff = eff.shape[0] if eff is not None else 0
    bin_k, cap = _build_bin_kernel(e_pad, shift, n_passes, ncls, n_eff)
    acc_k = _build_acc_kernel(cap, n_passes, ncls, R)
    if ncls == 3:
        seg_src, seg_dst, seg_ew, counts = bin_k(src, dst, ew, eff)
    else:
        seg_src, seg_dst, seg_ew, counts = bin_k(src, dst, ew)
    res = acc_k(x, seg_src, seg_dst, seg_ew, counts)
    return [r.reshape(-1, 16) if r.ndim == 1 else r for r in res]


def _fin1_kernel(s_ref, a_ref, w_ref, b_ref, o_ref):
    sw = a_ref[:, 0:1]
    cnt = a_ref[:, 1:2]
    lin = jnp.dot(s_ref[...], w_ref[...],
                  preferred_element_type=jnp.float32) + sw * b_ref[...]
    o_ref[...] = lin * jnp.where(cnt > 0, 1.0 / jnp.maximum(cnt, 1.0), 0.0)


def _finalize1(accs, W, b, n_out, blk=400):
    s, a = accs
    return pl.pallas_call(
        _fin1_kernel,
        grid=(n_out // blk,),
        in_specs=[
            pl.BlockSpec((blk, D), lambda i: (i, 0)),
            pl.BlockSpec((blk, 16), lambda i: (i, 0)),
            pl.BlockSpec((D, D), lambda i: (0, 0)),
            pl.BlockSpec((1, D), lambda i: (0, 0)),
        ],
        out_specs=pl.BlockSpec((blk, D), lambda i: (i, 0)),
        out_shape=jax.ShapeDtypeStruct((n_out, D), jnp.float32),
    )(s, a, W, b.reshape(1, D))


def _fin2_kernel(s1_ref, a1_ref, sp_ref, ap_ref, sn_ref, an_ref, sz_ref,
                 az_ref, w1_ref, b1_ref, w2_ref, b2_ref, wc_ref, wn_ref,
                 o_ref):
    sw = a1_ref[:, 0:1]
    cnt = a1_ref[:, 1:2]
    lin = jnp.dot(s1_ref[...], w1_ref[...],
                  preferred_element_type=jnp.float32) + sw * b1_ref[...]
    out = lin * jnp.where(cnt > 0, 1.0 / jnp.maximum(cnt, 1.0), 0.0)
    st = sp_ref[...] + sn_ref[...] + sz_ref[...]
    sw2 = ap_ref[:, 0:1] + an_ref[:, 0:1] + az_ref[:, 0:1]
    cnt2 = ap_ref[:, 1:2] + an_ref[:, 1:2] + az_ref[:, 1:2]
    lin2 = jnp.dot(st, w2_ref[...], preferred_element_type=jnp.float32) \
        + jnp.dot(sp_ref[...], wc_ref[...], preferred_element_type=jnp.float32) \
        - jnp.dot(sn_ref[...], wn_ref[...], preferred_element_type=jnp.float32) \
        + sw2 * b2_ref[...]
    out += lin2 * jnp.where(cnt2 > 0, 1.0 / jnp.maximum(cnt2, 1.0), 0.0)
    o_ref[...] = out


def _finalize2(acc1, acc3, W1, b1, W2, b2, Wc, Wn, n_out, blk=400):
    mat = pl.BlockSpec((D, D), lambda i: (0, 0))
    vec = pl.BlockSpec((1, D), lambda i: (0, 0))
    row = pl.BlockSpec((blk, D), lambda i: (i, 0))
    aux = pl.BlockSpec((blk, 16), lambda i: (i, 0))
    return pl.pallas_call(
        _fin2_kernel,
        grid=(n_out // blk,),
        in_specs=[row, aux, row, aux, row, aux, row, aux,
                  mat, vec, mat, vec, mat, mat],
        out_specs=pl.BlockSpec((blk, D), lambda i: (i, 0)),
        out_shape=jax.ShapeDtypeStruct((n_out, D), jnp.float32),
    )(acc1[0], acc1[1], acc3[0], acc3[1], acc3[2], acc3[3], acc3[4], acc3[5],
      W1, b1.reshape(1, D), W2, b2.reshape(1, D), Wc, Wn)


def kernel(x_word, x_topic, effect, src_ww, dst_ww, ew_ww, src_wt, dst_wt, ew_wt, src_wd, dst_wd, ew_wd, src_td, dst_td, ew_td, src_tt, dst_tt, ew_tt, W_ww, b_ww, W_wt, b_wt, W_wd, b_wd, W_td, b_td, W_tt, b_tt, W_causal, W_noise):
    acc_ww = _seg_reduce(x_word, src_ww, dst_ww, ew_ww, 100000,
                         13, 7, 1, 256)
    acc_wt = _seg_reduce(x_word, src_wt, dst_wt, ew_wt, 50000,
                         13, 2, 1, 256)
    acc_wd = _seg_reduce(x_word, src_wd, dst_wd, ew_wd, 100000,
                         13, 2, 1, 256)
    acc_td = _seg_reduce(x_topic, src_td, dst_td, ew_td, 30000,
                         11, 5, 3, 64, eff=effect)
    acc_tt = _seg_reduce(x_topic, src_tt, dst_tt, ew_tt, 30000,
                         11, 5, 3, 64, eff=effect)

    h_word = _finalize1(acc_ww, W_ww, b_ww, N_WORD)
    h_topic = _finalize2(acc_wt, acc_tt, W_wt, b_wt, W_tt, b_tt,
                         W_causal, W_noise, N_TOPIC)
    h_doc = _finalize2(acc_wd, acc_td, W_wd, b_wd, W_td, b_td,
                       W_causal, W_noise, N_DOC)
    return (h_word, h_topic, h_doc)


# speedup vs baseline: 43.8932x; 43.8932x over previous
import jax
import jax.numpy as jnp
from jax import lax
from jax.experimental import pallas as pl
from jax.experimental.pallas import tpu as pltpu
from jax.experimental.pallas import tpu_sc as plsc

D = 256
BE = 128
CH = 512
LANES = 16
SENT = 1 << 28
N_WORD, N_TOPIC, N_DOC = 50000, 10000, 10000


def _round_up(x, m):
    return (x + m - 1) // m * m


def _bi32(s):
    return jnp.broadcast_to(s.astype(jnp.int32), (LANES,))


def _build_bin_kernel(e_pad, shift, n_passes, ncls, n_eff=0):
    ept = e_pad // 32
    nbins = n_passes * ncls
    cap = ept + CH + 16

    mesh = plsc.VectorSubcoreMesh(core_axis_name="c", subcore_axis_name="s")

    scratch = [
        pltpu.VMEM((ept,), jnp.int32),
        pltpu.VMEM((ept,), jnp.int32),
        pltpu.VMEM((ept,), jnp.float32),
        pltpu.VMEM((16,), jnp.int32),
    ]
    for _ in range(nbins):
        scratch += [
            pltpu.VMEM((cap,), jnp.int32),
            pltpu.VMEM((cap,), jnp.int32),
            pltpu.VMEM((cap,), jnp.float32),
        ]
    if ncls == 3:
        scratch.append(pltpu.VMEM((n_eff,), jnp.float32))
    scratch.append(pltpu.SemaphoreType.DMA)

    def body(*refs):
        nin = 3 + (1 if ncls == 3 else 0)
        src_hbm, dst_hbm, ew_hbm = refs[0:3]
        eff_hbm = refs[3] if ncls == 3 else None
        seg_src, seg_dst, seg_ew, counts_hbm = refs[nin:nin + 4]
        rest = refs[nin + 4:]
        src_v, dst_v, ew_v, counts_buf = rest[0:4]
        cmp = [rest[4 + 3 * b:7 + 3 * b] for b in range(nbins)]
        if ncls == 3:
            eff_v = rest[4 + 3 * nbins]
        sem = rest[-1]

        core = lax.axis_index("c")
        tile = lax.axis_index("s")
        wid = core * 16 + tile

        pltpu.sync_copy(src_hbm.at[pl.ds(wid * ept, ept)], src_v)
        pltpu.sync_copy(dst_hbm.at[pl.ds(wid * ept, ept)], dst_v)
        pltpu.sync_copy(ew_hbm.at[pl.ds(wid * ept, ept)], ew_v)
        if ncls == 3:
            pltpu.sync_copy(eff_hbm, eff_v)

        lane = lax.broadcasted_iota(jnp.int32, (LANES,), 0)
        fzero_v = jnp.full((LANES,), 0.0, jnp.float32)
        trash_v = jnp.full((LANES,), cap - 1, jnp.int32)
        sent_v = jnp.full((LANES,), SENT, jnp.int32)

        def scan_body(i, cnts):
            b = i * LANES
            sv = src_v[pl.ds(b, LANES)]
            dv = dst_v[pl.ds(b, LANES)]
            wv = ew_v[pl.ds(b, LANES)]
            pv = lax.shift_right_logical(dv, jnp.full((LANES,), shift,
                                                      jnp.int32))
            if ncls == 3:
                ev = plsc.load_gather(eff_v, [sv])
                clsv = jnp.where(ev > fzero_v, _bi32(jnp.int32(0)),
                                 jnp.where(ev < fzero_v,
                                           _bi32(jnp.int32(1)),
                                           _bi32(jnp.int32(2))))
                binv = pv * _bi32(jnp.int32(3)) + clsv
            else:
                binv = pv
            out = []
            for b_i in range(nbins):
                m = binv == _bi32(jnp.int32(b_i))
                mi = m.astype(jnp.int32)
                rank = plsc.cumsum(mi) - mi
                pos = jnp.where(m, _bi32(cnts[b_i]) + rank, trash_v)
                plsc.store_scatter(cmp[b_i][0], [pos], sv)
                plsc.store_scatter(cmp[b_i][1], [pos], dv)
                plsc.store_scatter(cmp[b_i][2], [pos], wv)
                out.append(cnts[b_i] + jnp.sum(mi))
            return tuple(out)

        cnts = lax.fori_loop(0, ept // LANES, scan_body,
                             tuple(jnp.int32(0) for _ in range(nbins)))

        for b_i in range(nbins):
            for k in range(CH // LANES):
                pos = _bi32(cnts[b_i]) + lane + k * LANES
                plsc.store_scatter(cmp[b_i][1], [pos], sent_v)
            plsc.store_scatter(counts_buf, [_bi32(jnp.int32(b_i))],
                               _bi32(cnts[b_i]))
            seg_off = (b_i * 32 + wid) * cap
            pltpu.sync_copy(cmp[b_i][0], seg_src.at[pl.ds(seg_off, cap)])
            pltpu.sync_copy(cmp[b_i][1], seg_dst.at[pl.ds(seg_off, cap)])
            pltpu.sync_copy(cmp[b_i][2], seg_ew.at[pl.ds(seg_off, cap)])
        pltpu.sync_copy(counts_buf, counts_hbm.at[pl.ds(wid * 16, 16)])

    out_type = [
        jax.ShapeDtypeStruct((nbins * 32 * cap,), jnp.int32),
        jax.ShapeDtypeStruct((nbins * 32 * cap,), jnp.int32),
        jax.ShapeDtypeStruct((nbins * 32 * cap,), jnp.float32),
        jax.ShapeDtypeStruct((512,), jnp.int32),
    ]
    return pl.kernel(body, out_type=out_type, mesh=mesh,
                     scratch_types=scratch,
                     compiler_params=pltpu.CompilerParams(
                         needs_layout_passes=False)), cap


def _build_acc_kernel(seg_cap, n_passes, ncls, R):
    n_dst_pad = n_passes * 32 * R
    mesh = plsc.VectorSubcoreMesh(core_axis_name="c", subcore_axis_name="s")

    ccap = CH + BE + 16
    scratch = [
        pltpu.VMEM((CH,), jnp.int32),
        pltpu.VMEM((CH,), jnp.int32),
        pltpu.VMEM((CH,), jnp.float32),
        pltpu.VMEM((ccap,), jnp.int32),
        pltpu.VMEM((ccap,), jnp.int32),
        pltpu.VMEM((ccap,), jnp.float32),
        pltpu.VMEM((BE,), jnp.int32),
        pltpu.VMEM((BE,), jnp.int32),
        pltpu.VMEM((BE, D), jnp.float32),
        pltpu.VMEM((512,), jnp.int32),
    ]
    for _ in range(ncls):
        scratch.append(pltpu.VMEM((R + 8, D), jnp.float32))
    for _ in range(ncls):
        scratch.append(pltpu.VMEM(((R + 8) * 16,), jnp.float32))
    scratch.append(pltpu.SemaphoreType.DMA)

    def body(*refs):
        x_hbm, seg_src, seg_dst, seg_ew, counts_hbm = refs[0:5]
        outs = refs[5:5 + 2 * ncls]
        rest = refs[5 + 2 * ncls:]
        (sbuf_src, sbuf_dst, sbuf_ew, cmp_src, cmp_dl, cmp_ew,
         src_blk, dl_blk, stage, counts_v) = rest[0:10]
        accs = rest[10:10 + ncls]
        auxs = rest[10 + ncls:10 + 2 * ncls]
        sem = rest[-1]

        core = lax.axis_index("c")
        tile = lax.axis_index("s")
        wid = core * 16 + tile

        pltpu.sync_copy(counts_hbm, counts_v)

        lane = lax.broadcasted_iota(jnp.int32, (LANES,), 0)
        blend0 = lane == 0
        blend1 = lane == 1
        ones_v = jnp.full((LANES,), 1.0, jnp.float32)
        fzero_v = jnp.full((LANES,), 0.0, jnp.float32)
        zero_iv = jnp.full((LANES,), 0, jnp.int32)
        trash_v = jnp.full((LANES,), ccap - 1, jnp.int32)
        dummy_v = jnp.full((LANES,), R, jnp.int32)
        cols = [lane + g * LANES for g in range(D // LANES)]

        def _scal(ref, idx):
            g = plsc.load_gather(ref, [_bi32(idx)])
            return jnp.sum(jnp.where(blend0, g, zero_iv))

        def pass_body(p, _):
            my_lo = (p * 32 + wid) * R
            lo_v = _bi32(my_lo)
            hi_v = _bi32(my_lo + R)

            def zero_body(r, _):
                for g in range(D // LANES):
                    accs_r = r
                    for c in range(ncls):
                        accs[c][accs_r, pl.ds(g * LANES, LANES)] = fzero_v
                for c in range(ncls):
                    auxs[c][pl.ds(r * LANES, LANES)] = fzero_v
                return 0

            lax.fori_loop(0, R, zero_body, 0)

            for cls in range(ncls):
                b_i = p * ncls + cls
                acc = accs[cls]
                aux = auxs[cls]

                def do_rows(base, nrows, acc=acc, aux=aux):
                    pltpu.async_copy(
                        x_hbm.at[cmp_src.at[pl.ds(base, BE)]], stage,
                        sem).wait()

                    def row_body(j, _):
                        rowv = plsc.load_gather(cmp_dl, [_bi32(base + j)])
                        ewv = plsc.load_gather(cmp_ew, [_bi32(base + j)])
                        for g in range(D // LANES):
                            v = stage[j, pl.ds(g * LANES, LANES)]
                            plsc.addupdate_scatter(
                                acc, [rowv, cols[g]], v * ewv)
                        av = jnp.where(
                            blend0, ewv,
                            jnp.where(blend1, ones_v, fzero_v))
                        plsc.addupdate_scatter(
                            aux, [rowv * LANES + lane], av)
                        return 0

                    lax.fori_loop(0, nrows, row_body, 0)

                def w_body(w, rem, b_i=b_i, acc=acc, aux=aux,
                           lo_v=lo_v, hi_v=hi_v):
                    cnt = _scal(counts_v, w * 16 + b_i)
                    nch = (cnt + (CH - 1)) // CH

                    def ch_body(ch, rem, w=w, b_i=b_i, acc=acc, aux=aux,
                                cnt=cnt, lo_v=lo_v, hi_v=hi_v):
                        off = (b_i * 32 + w) * seg_cap + ch * CH
                        d1 = pltpu.async_copy(
                            seg_src.at[pl.ds(off, CH)], sbuf_src, sem)
                        d2 = pltpu.async_copy(
                            seg_dst.at[pl.ds(off, CH)], sbuf_dst, sem)
                        d3 = pltpu.async_copy(
                            seg_ew.at[pl.ds(off, CH)], sbuf_ew, sem)
                        d1.wait(); d2.wait(); d3.wait()
                        nv = jnp.minimum((cnt - ch * CH + (LANES - 1))
                                         // LANES, CH // LANES)

                        def scan_body(i, cnt2):
                            b = i * LANES
                            sv = sbuf_src[pl.ds(b, LANES)]
                            dv = sbuf_dst[pl.ds(b, LANES)]
                            wv = sbuf_ew[pl.ds(b, LANES)]
                            m = (dv >= lo_v) & (dv < hi_v)
                            mi = m.astype(jnp.int32)
                            rank = plsc.cumsum(mi) - mi
                            pos = jnp.where(m, _bi32(cnt2) + rank, trash_v)
                            plsc.store_scatter(cmp_src, [pos], sv)
                            plsc.store_scatter(cmp_dl, [pos], dv - lo_v)
                            plsc.store_scatter(cmp_ew, [pos], wv)
                            return cnt2 + jnp.sum(mi)

                        new_len = lax.fori_loop(0, nv, scan_body, rem)
                        nfull = new_len // BE

                        def blk_body(b2, _, acc=acc, aux=aux):
                            do_rows(b2 * BE, BE, acc=acc, aux=aux)
                            return 0

                        lax.fori_loop(0, nfull, blk_body, 0)

                        sh = nfull * BE
                        for g in range(BE // LANES):
                            o2 = g * LANES
                            sv = cmp_src[pl.ds(sh + o2, LANES)]
                            dv = cmp_dl[pl.ds(sh + o2, LANES)]
                            wv = cmp_ew[pl.ds(sh + o2, LANES)]
                            cmp_src[pl.ds(o2, LANES)] = sv
                            cmp_dl[pl.ds(o2, LANES)] = dv
                            cmp_ew[pl.ds(o2, LANES)] = wv
                        return new_len - nfull * BE

                    return lax.fori_loop(0, nch, ch_body, rem)

                rem = lax.fori_loop(0, 32, w_body, jnp.int32(0))

                @pl.when(rem > 0)
                def _(rem=rem, acc=acc, aux=aux):
                    for g in range(BE // LANES):
                        o2 = g * LANES
                        sv = cmp_src[pl.ds(o2, LANES)]
                        dv = cmp_dl[pl.ds(o2, LANES)]
                        valid = (lane + _bi32(jnp.int32(o2))) < _bi32(rem)
                        cmp_src[pl.ds(o2, LANES)] = jnp.where(
                            valid, sv, zero_iv)
                        cmp_dl[pl.ds(o2, LANES)] = jnp.where(
                            valid, dv, dummy_v)
                    do_rows(jnp.int32(0), rem, acc=acc, aux=aux)

            for c in range(ncls):
                pltpu.sync_copy(accs[c].at[pl.ds(0, R)],
                                outs[2 * c].at[pl.ds(my_lo, R)])
                pltpu.sync_copy(auxs[c].at[pl.ds(0, R * 16)],
                                outs[2 * c + 1].at[pl.ds(my_lo * 16, R * 16)])
            return 0

        lax.fori_loop(0, n_passes, pass_body, 0)

    out_type = []
    for _ in range(ncls):
        out_type += [jax.ShapeDtypeStruct((n_dst_pad, D), jnp.float32),
                     jax.ShapeDtypeStruct((n_dst_pad * 16,), jnp.float32)]
    return pl.kernel(body, out_type=out_type, mesh=mesh,
                     scratch_types=scratch,
                     compiler_params=pltpu.CompilerParams(
                         needs_layout_passes=False))


def _seg_reduce(x, src, dst, ew, E, shift, n_passes, ncls, R, eff=None):
    e_pad = _round_up(E, 512)
    p = e_pad - E
    src = jnp.pad(src, (0, p))
    dst = jnp.pad(dst, (0, p), constant_values=SENT)
    ew = jnp.pad(ew, (0, p))
    n_eff = eff.shape[0] if eff is not None else 0
    bin_k, cap = _build_bin_kernel(e_pad, shift, n_passes, ncls, n_eff)
    acc_k = _build_acc_kernel(cap, n_passes, ncls, R)
    if ncls == 3:
        seg_src, seg_dst, seg_ew, counts = bin_k(src, dst, ew, eff)
    else:
        seg_src, seg_dst, seg_ew, counts = bin_k(src, dst, ew)
    res = acc_k(x, seg_src, seg_dst, seg_ew, counts)
    return [r.reshape(-1, 16) if r.ndim == 1 else r for r in res]


def _fin1_kernel(s_ref, a_ref, w_ref, b_ref, o_ref):
    sw = a_ref[:, 0:1]
    cnt = a_ref[:, 1:2]
    lin = jnp.dot(s_ref[...], w_ref[...],
                  preferred_element_type=jnp.float32) + sw * b_ref[...]
    o_ref[...] = lin * jnp.where(cnt > 0, 1.0 / jnp.maximum(cnt, 1.0), 0.0)


def _finalize1(accs, W, b, n_out, blk=400):
    s, a = accs
    return pl.pallas_call(
        _fin1_kernel,
        grid=(n_out // blk,),
        in_specs=[
            pl.BlockSpec((blk, D), lambda i: (i, 0)),
            pl.BlockSpec((blk, 16), lambda i: (i, 0)),
            pl.BlockSpec((D, D), lambda i: (0, 0)),
            pl.BlockSpec((1, D), lambda i: (0, 0)),
        ],
        out_specs=pl.BlockSpec((blk, D), lambda i: (i, 0)),
        out_shape=jax.ShapeDtypeStruct((n_out, D), jnp.float32),
    )(s, a, W, b.reshape(1, D))


def _fin2_kernel(s1_ref, a1_ref, sp_ref, ap_ref, sn_ref, an_ref, sz_ref,
                 az_ref, w1_ref, b1_ref, w2_ref, b2_ref, wc_ref, wn_ref,
                 o_ref):
    sw = a1_ref[:, 0:1]
    cnt = a1_ref[:, 1:2]
    lin = jnp.dot(s1_ref[...], w1_ref[...],
                  preferred_element_type=jnp.float32) + sw * b1_ref[...]
    out = lin * jnp.where(cnt > 0, 1.0 / jnp.maximum(cnt, 1.0), 0.0)
    st = sp_ref[...] + sn_ref[...] + sz_ref[...]
    sw2 = ap_ref[:, 0:1] + an_ref[:, 0:1] + az_ref[:, 0:1]
    cnt2 = ap_ref[:, 1:2] + an_ref[:, 1:2] + az_ref[:, 1:2]
    lin2 = jnp.dot(st, w2_ref[...], preferred_element_type=jnp.float32) \
        + jnp.dot(sp_ref[...], wc_ref[...], preferred_element_type=jnp.float32) \
        - jnp.dot(sn_ref[...], wn_ref[...], preferred_element_type=jnp.float32) \
        + sw2 * b2_ref[...]
    out += lin2 * jnp.where(cnt2 > 0, 1.0 / jnp.maximum(cnt2, 1.0), 0.0)
    o_ref[...] = out


def _finalize2(acc1, acc3, W1, b1, W2, b2, Wc, Wn, n_out, blk=400):
    mat = pl.BlockSpec((D, D), lambda i: (0, 0))
    vec = pl.BlockSpec((1, D), lambda i: (0, 0))
    row = pl.BlockSpec((blk, D), lambda i: (i, 0))
    aux = pl.BlockSpec((blk, 16), lambda i: (i, 0))
    return pl.pallas_call(
        _fin2_kernel,
        grid=(n_out // blk,),
        in_specs=[row, aux, row, aux, row, aux, row, aux,
                  mat, vec, mat, vec, mat, mat],
        out_specs=pl.BlockSpec((blk, D), lambda i: (i, 0)),
        out_shape=jax.ShapeDtypeStruct((n_out, D), jnp.float32),
    )(acc1[0], acc1[1], acc3[0], acc3[1], acc3[2], acc3[3], acc3[4], acc3[5],
      W1, b1.reshape(1, D), W2, b2.reshape(1, D), Wc, Wn)


def kernel(x_word, x_topic, effect, src_ww, dst_ww, ew_ww, src_wt, dst_wt, ew_wt, src_wd, dst_wd, ew_wd, src_td, dst_td, ew_td, src_tt, dst_tt, ew_tt, W_ww, b_ww, W_wt, b_wt, W_wd, b_wd, W_td, b_td, W_tt, b_tt, W_causal, W_noise):
    acc_ww = _seg_reduce(x_word, src_ww, dst_ww, ew_ww, 100000,
                         13, 7, 1, 256)
    acc_wt = _seg_reduce(x_word, src_wt, dst_wt, ew_wt, 50000,
                         13, 2, 1, 256)
    acc_wd = _seg_reduce(x_word, src_wd, dst_wd, ew_wd, 100000,
                         13, 2, 1, 256)
    acc_td = _seg_reduce(x_topic, src_td, dst_td, ew_td, 30000,
                         11, 5, 3, 64, eff=effect)
    acc_tt = _seg_reduce(x_topic, src_tt, dst_tt, ew_tt, 30000,
                         11, 5, 3, 64, eff=effect)

    h_word = _finalize1(acc_ww, W_ww, b_ww, N_WORD)
    h_topic = _finalize2(acc_wt, acc_tt, W_wt, b_wt, W_tt, b_tt,
                         W_causal, W_noise, N_TOPIC)
    h_doc = _finalize2(acc_wd, acc_td, W_wd, b_wd, W_td, b_td,
                       W_causal, W_noise, N_DOC)
    return (h_word, h_topic, h_doc)
